# Initial kernel scaffold; baseline (speedup 1.0000x reference)
import jax
import jax.numpy as jnp
from jax.experimental import pallas as pl


def kernel(x, edge_index, params):
    raise NotImplementedError("write your pallas kernel here")


# speedup vs baseline: 7.3748x; 7.3748x over previous
import functools

import jax
import jax.numpy as jnp
from jax import lax
from jax.experimental import pallas as pl
from jax.experimental.pallas import tpu as pltpu
from jax.experimental.pallas import tpu_sc as plsc

N = 10000
E = 160000
H = 256
HALF = 128
NC = 2
NS = 16
LANES = 16

CHUNK = 128
CHUNKS_ROW = 80
E_PAD = NS * CHUNKS_ROW * CHUNK
CHUNKS_W = E_PAD // (NC * NS * CHUNK)
ACC_R = 10240
ZSL = ACC_R // NS
RSL = N // NS
RCH = 125
EPS = 1e-5

def _mesh():
  return plsc.VectorSubcoreMesh(core_axis_name="core", subcore_axis_name="subcore")


_SC_PARAMS = pltpu.CompilerParams(use_tc_tiling_on_sc=False)


def _fill(ref, nrows, width, value):
  vec = jnp.full((LANES,), value, jnp.float32)

  @pl.loop(0, nrows)
  def _(i):
    for j in range(width // LANES):
      ref[i, pl.ds(j * LANES, LANES)] = vec


def _sc_hist16_body(gather_xs, xs_hbm, src_hbm, dst_hbm, out_hbm,
                    src_v, dst_v, rows_v, stage_v, acc_sh):
  c = lax.axis_index("core")
  s = lax.axis_index("subcore")
  if gather_xs:
    pltpu.sync_copy(src_hbm.at[c, s], src_v)
  pltpu.sync_copy(dst_hbm.at[c, s], dst_v)
  _fill(stage_v, CHUNK, LANES, 0.0)
  @pl.loop(0, ZSL // CHUNK)
  def _(k):
    pltpu.sync_copy(stage_v, acc_sh.at[pl.ds(s * ZSL + k * CHUNK, CHUNK)])
  if not gather_xs:
    _fill(rows_v, CHUNK, LANES, 1.0)
  plsc.subcore_barrier()

  @pl.loop(0, CHUNKS_W)
  def _(j):
    if gather_xs:
      pltpu.sync_copy(xs_hbm.at[src_v.at[j]], rows_v)
    pltpu.sync_copy(rows_v, acc_sh.at[dst_v.at[j]], add=True)

  plsc.subcore_barrier()
  @pl.loop(0, ZSL // CHUNK)
  def _(k):
    base = s * ZSL + k * CHUNK
    pltpu.sync_copy(acc_sh.at[pl.ds(base, CHUNK)], stage_v)
    pltpu.sync_copy(stage_v, out_hbm.at[c].at[pl.ds(base, CHUNK)])


def _sc_rows_body(table_hbm, src_hbm, dst_hbm, out_hbm,
                  src_v, dst_v, rows_v, acc_sh):
  c = lax.axis_index("core")
  s = lax.axis_index("subcore")
  pltpu.sync_copy(src_hbm.at[s], src_v)
  pltpu.sync_copy(dst_hbm.at[s], dst_v)
  _fill(rows_v, CHUNK, HALF, 0.0)
  @pl.loop(0, ZSL // CHUNK)
  def _(k):
    pltpu.sync_copy(rows_v, acc_sh.at[pl.ds(s * ZSL + k * CHUNK, CHUNK)])
  plsc.subcore_barrier()

  @pl.loop(0, CHUNKS_ROW)
  def _(j):
    pltpu.sync_copy(table_hbm.at[c].at[src_v.at[j]], rows_v)
    pltpu.sync_copy(rows_v, acc_sh.at[dst_v.at[j]], add=True)

  plsc.subcore_barrier()
  @pl.loop(0, ZSL // CHUNK)
  def _(k):
    base = s * ZSL + k * CHUNK
    pltpu.sync_copy(acc_sh.at[pl.ds(base, CHUNK)], rows_v)
    pltpu.sync_copy(rows_v, out_hbm.at[c].at[pl.ds(base, CHUNK)])


def _sc_hist16(gather_xs, xs16, src_w, dst_w):
  kern = pl.kernel(
      functools.partial(_sc_hist16_body, gather_xs),
      out_type=jax.ShapeDtypeStruct((NC, ACC_R, LANES), jnp.float32),
      mesh=_mesh(),
      compiler_params=_SC_PARAMS,
      scratch_types=[
          pltpu.VMEM((CHUNKS_W, CHUNK), jnp.int32),
          pltpu.VMEM((CHUNKS_W, CHUNK), jnp.int32),
          pltpu.VMEM((CHUNK, LANES), jnp.float32),
          pltpu.VMEM((CHUNK, LANES), jnp.float32),
          pltpu.VMEM_SHARED((ACC_R, LANES), jnp.float32),
      ],
  )
  return kern(xs16, src_w, dst_w)


def _sc_rows(table, src_r, dst_r):
  kern = pl.kernel(
      _sc_rows_body,
      out_type=jax.ShapeDtypeStruct((NC, ACC_R, HALF), jnp.float32),
      mesh=_mesh(),
      compiler_params=_SC_PARAMS,
      scratch_types=[
          pltpu.VMEM((CHUNKS_ROW, CHUNK), jnp.int32),
          pltpu.VMEM((CHUNKS_ROW, CHUNK), jnp.int32),
          pltpu.VMEM((CHUNK, HALF), jnp.float32),
          pltpu.VMEM_SHARED((ACC_R, HALF), jnp.float32),
      ],
  )
  return kern(table, src_r, dst_r)


def _bn_relu(pre, gamma, beta):
  mean = jnp.mean(pre, axis=0, keepdims=True)
  var = jnp.mean((pre - mean) ** 2, axis=0, keepdims=True)
  return jnp.maximum(gamma * (pre - mean) * lax.rsqrt(var + EPS) + beta, 0.0)


def _tc_prep_body(deg2_ref, x_ref, dinv_ref, xs16_ref):
  deg = deg2_ref[0, :N, 0:1] + deg2_ref[1, :N, 0:1] + 1.0
  dinv = lax.rsqrt(deg)
  dinv_ref[...] = dinv
  xs16_ref[...] = jnp.broadcast_to(dinv * x_ref[...], (N, LANES))


def _tc_layer0_body(s0_ref, xs16_ref, dinv_ref, w0_ref, b0_ref, g0_ref,
                    be0_ref, w1_ref, hs_ref):
  dinv = dinv_ref[...]
  t = dinv * (s0_ref[0, :N, 0:1] + s0_ref[1, :N, 0:1] + xs16_ref[:, 0:1])
  pre = t * w0_ref[...] + b0_ref[...]
  h = _bn_relu(pre, g0_ref[...], be0_ref[...])
  hw = jnp.dot(h, w1_ref[...], preferred_element_type=jnp.float32)
  hs = dinv * hw
  hs_ref[0] = hs[:, :HALF]
  hs_ref[1] = hs[:, HALF:]


def _tc_mid_body(acc_ref, hsp_ref, dinv_ref, b_ref, g_ref, be_ref, w_ref,
                 hs_ref):
  dinv = dinv_ref[...]
  h_halves = []
  for c in range(NC):
    pre = dinv * (acc_ref[c, :N] + hsp_ref[c]) + b_ref[:, c * HALF:(c + 1) * HALF]
    h_halves.append(_bn_relu(pre, g_ref[:, c * HALF:(c + 1) * HALF],
                             be_ref[:, c * HALF:(c + 1) * HALF]))
  hw = (jnp.dot(h_halves[0], w_ref[:HALF, :], preferred_element_type=jnp.float32)
        + jnp.dot(h_halves[1], w_ref[HALF:, :], preferred_element_type=jnp.float32))
  hs = dinv * hw
  hs_ref[0] = hs[:, :HALF]
  hs_ref[1] = hs[:, HALF:]


def _tc_final_body(acc_ref, hsp_ref, dinv_ref, b_ref, g_ref, be_ref,
                   l1w_ref, l1b_ref, l2w_ref, l2b_ref, out_ref):
  dinv = dinv_ref[...]
  g_halves = []
  for c in range(NC):
    pre = dinv * (acc_ref[c, :N] + hsp_ref[c]) + b_ref[:, c * HALF:(c + 1) * HALF]
    h = _bn_relu(pre, g_ref[:, c * HALF:(c + 1) * HALF],
                 be_ref[:, c * HALF:(c + 1) * HALF])
    g_halves.append(jnp.mean(h, axis=0, keepdims=True))
  g1 = (jnp.dot(g_halves[0], l1w_ref[:HALF, :], preferred_element_type=jnp.float32)
        + jnp.dot(g_halves[1], l1w_ref[HALF:, :], preferred_element_type=jnp.float32))
  g1 = jnp.maximum(g1 + l1b_ref[...], 0.0)
  out_ref[...] = jnp.dot(g1, l2w_ref[...], preferred_element_type=jnp.float32) + l2b_ref[...]


def _tc(body, out_shapes, *args):
  return pl.pallas_call(
      body, out_shape=out_shapes,
      compiler_params=pltpu.CompilerParams(vmem_limit_bytes=64 * 1024 * 1024),
  )(*args)


def kernel(x, edge_index, params):
  src = edge_index[0].astype(jnp.int32)
  dst = edge_index[1].astype(jnp.int32)
  pad = E_PAD - E
  src_p = jnp.concatenate([src, jnp.zeros((pad,), jnp.int32)])
  dst_p = jnp.concatenate([dst, jnp.full((pad,), N, jnp.int32)])
  src_w = src_p.reshape(NC, NS, CHUNKS_W, CHUNK)
  dst_w = dst_p.reshape(NC, NS, CHUNKS_W, CHUNK)
  src_r = src_p.reshape(NS, CHUNKS_ROW, CHUNK)
  dst_r = dst_p.reshape(NS, CHUNKS_ROW, CHUNK)

  convs = params["convs"]
  bns = params["bns"]
  w0 = convs[0]["W"].reshape(1, H)
  bs = [c["b"].reshape(1, H) for c in convs]
  gs = [bn["gamma"].reshape(1, H) for bn in bns]
  bes = [bn["beta"].reshape(1, H) for bn in bns]
  ws = [convs[l]["W"] for l in range(1, 4)]

  deg2 = _sc_hist16(False, jnp.zeros((N, LANES), jnp.float32), src_w, dst_w)
  dinv, xs16 = _tc(
      _tc_prep_body,
      (jax.ShapeDtypeStruct((N, 1), jnp.float32),
       jax.ShapeDtypeStruct((N, LANES), jnp.float32)),
      deg2, x)

  s0 = _sc_hist16(True, xs16, src_w, dst_w)
  hs = _tc(
      _tc_layer0_body,
      jax.ShapeDtypeStruct((NC, N, HALF), jnp.float32),
      s0, xs16, dinv, w0, bs[0], gs[0], bes[0], ws[0])

  for l in (1, 2):
    acc = _sc_rows(hs, src_r, dst_r)
    hs = _tc(
        _tc_mid_body,
        jax.ShapeDtypeStruct((NC, N, HALF), jnp.float32),
        acc, hs, dinv, bs[l], gs[l], bes[l], ws[l])

  acc = _sc_rows(hs, src_r, dst_r)
  out = _tc(
      _tc_final_body,
      jax.ShapeDtypeStruct((1, 1), jnp.float32),
      acc, hs, dinv, bs[3], gs[3], bes[3],
      params["lin1_w"], params["lin1_b"].reshape(1, H),
      params["lin2_w"], params["lin2_b"].reshape(1, 1))
  return out


# speedup vs baseline: 7.5233x; 1.0201x over previous
import functools

import jax
import jax.numpy as jnp
from jax import lax
from jax.experimental import pallas as pl
from jax.experimental.pallas import tpu as pltpu
from jax.experimental.pallas import tpu_sc as plsc

N = 10000
E = 160000
H = 256
HALF = 128
NC = 2
NS = 16
LANES = 16

CHUNK = 128
CHUNKS_ROW = 80
E_PAD = NS * CHUNKS_ROW * CHUNK
CH2 = 64
CHUNKS2 = E_PAD // (NS * CH2)
CHUNKS_W = E_PAD // (NC * NS * CHUNK)
ACC_R = 10240
ZSL = ACC_R // NS
RSL = N // NS
RCH = 125
EPS = 1e-5

def _mesh():
  return plsc.VectorSubcoreMesh(core_axis_name="core", subcore_axis_name="subcore")


_SC_PARAMS = pltpu.CompilerParams(use_tc_tiling_on_sc=False)


def _fill(ref, nrows, width, value):
  vec = jnp.full((LANES,), value, jnp.float32)

  @pl.loop(0, nrows)
  def _(i):
    for j in range(width // LANES):
      ref[i, pl.ds(j * LANES, LANES)] = vec


def _sc_hist16_body(gather_xs, xs_hbm, src_hbm, dst_hbm, out_hbm,
                    src_v, dst_v, rows_v, stage_v, acc_sh):
  c = lax.axis_index("core")
  s = lax.axis_index("subcore")
  if gather_xs:
    pltpu.sync_copy(src_hbm.at[c, s], src_v)
  pltpu.sync_copy(dst_hbm.at[c, s], dst_v)
  _fill(stage_v, CHUNK, LANES, 0.0)
  @pl.loop(0, ZSL // CHUNK)
  def _(k):
    pltpu.sync_copy(stage_v, acc_sh.at[pl.ds(s * ZSL + k * CHUNK, CHUNK)])
  if not gather_xs:
    _fill(rows_v, CHUNK, LANES, 1.0)
  plsc.subcore_barrier()

  @pl.loop(0, CHUNKS_W)
  def _(j):
    if gather_xs:
      pltpu.sync_copy(xs_hbm.at[src_v.at[j]], rows_v)
    pltpu.sync_copy(rows_v, acc_sh.at[dst_v.at[j]], add=True)

  plsc.subcore_barrier()
  @pl.loop(0, ZSL // CHUNK)
  def _(k):
    base = s * ZSL + k * CHUNK
    pltpu.sync_copy(acc_sh.at[pl.ds(base, CHUNK)], stage_v)
    pltpu.sync_copy(stage_v, out_hbm.at[c].at[pl.ds(base, CHUNK)])


def _sc_rows_body(table_hbm, src_hbm, dst_hbm, out_hbm,
                  src_v, dst_v, rows0, rows1, gs0, gs1, ss0, ss1, acc_sh):
  c = lax.axis_index("core")
  s = lax.axis_index("subcore")
  pltpu.sync_copy(src_hbm.at[s], src_v)
  pltpu.sync_copy(dst_hbm.at[s], dst_v)
  _fill(rows0, CH2, HALF, 0.0)
  @pl.loop(0, ZSL // CH2)
  def _(k):
    pltpu.sync_copy(rows0, acc_sh.at[pl.ds(s * ZSL + k * CH2, CH2)])
  plsc.subcore_barrier()

  def g_start(j, buf, sem):
    pltpu.async_copy(table_hbm.at[c].at[src_v.at[j]], buf, sem)

  def g_wait(j, buf, sem):
    pltpu.make_async_copy(table_hbm.at[c].at[src_v.at[j]], buf, sem).wait()

  def s_start(j, buf, sem):
    pltpu.async_copy(buf, acc_sh.at[dst_v.at[j]], sem, add=True)

  def s_wait(j, buf, sem):
    pltpu.make_async_copy(buf, acc_sh.at[dst_v.at[j]], sem).wait()

  g_start(0, rows0, gs0)
  g_wait(0, rows0, gs0)
  g_start(1, rows1, gs1)
  s_start(0, rows0, ss0)
  g_wait(1, rows1, gs1)
  s_wait(0, rows0, ss0)
  g_start(2, rows0, gs0)
  s_start(1, rows1, ss1)

  @pl.loop(1, CHUNKS2 // 2 - 1)
  def _(k):
    j = 2 * k
    g_wait(j, rows0, gs0)
    s_wait(j - 1, rows1, ss1)
    g_start(j + 1, rows1, gs1)
    s_start(j, rows0, ss0)
    g_wait(j + 1, rows1, gs1)
    s_wait(j, rows0, ss0)
    g_start(j + 2, rows0, gs0)
    s_start(j + 1, rows1, ss1)

  jl = CHUNKS2 - 2
  g_wait(jl, rows0, gs0)
  s_wait(jl - 1, rows1, ss1)
  g_start(jl + 1, rows1, gs1)
  s_start(jl, rows0, ss0)
  g_wait(jl + 1, rows1, gs1)
  s_start(jl + 1, rows1, ss1)
  s_wait(jl, rows0, ss0)
  s_wait(jl + 1, rows1, ss1)

  plsc.subcore_barrier()
  @pl.loop(0, ZSL // CH2)
  def _(k):
    base = s * ZSL + k * CH2
    pltpu.sync_copy(acc_sh.at[pl.ds(base, CH2)], rows0)
    pltpu.sync_copy(rows0, out_hbm.at[c].at[pl.ds(base, CH2)])


def _sc_hist16(gather_xs, xs16, src_w, dst_w):
  kern = pl.kernel(
      functools.partial(_sc_hist16_body, gather_xs),
      out_type=jax.ShapeDtypeStruct((NC, ACC_R, LANES), jnp.float32),
      mesh=_mesh(),
      compiler_params=_SC_PARAMS,
      scratch_types=[
          pltpu.VMEM((CHUNKS_W, CHUNK), jnp.int32),
          pltpu.VMEM((CHUNKS_W, CHUNK), jnp.int32),
          pltpu.VMEM((CHUNK, LANES), jnp.float32),
          pltpu.VMEM((CHUNK, LANES), jnp.float32),
          pltpu.VMEM_SHARED((ACC_R, LANES), jnp.float32),
      ],
  )
  return kern(xs16, src_w, dst_w)


def _sc_rows(table, src_r, dst_r):
  kern = pl.kernel(
      _sc_rows_body,
      out_type=jax.ShapeDtypeStruct((NC, ACC_R, HALF), jnp.float32),
      mesh=_mesh(),
      compiler_params=_SC_PARAMS,
      scratch_types=[
          pltpu.VMEM((CHUNKS2, CH2), jnp.int32),
          pltpu.VMEM((CHUNKS2, CH2), jnp.int32),
          pltpu.VMEM((CH2, HALF), jnp.float32),
          pltpu.VMEM((CH2, HALF), jnp.float32),
          pltpu.SemaphoreType.DMA,
          pltpu.SemaphoreType.DMA,
          pltpu.SemaphoreType.DMA,
          pltpu.SemaphoreType.DMA,
          pltpu.VMEM_SHARED((ACC_R, HALF), jnp.float32),
      ],
  )
  return kern(table, src_r, dst_r)


def _bn_relu(pre, gamma, beta):
  mean = jnp.mean(pre, axis=0, keepdims=True)
  var = jnp.mean((pre - mean) ** 2, axis=0, keepdims=True)
  return jnp.maximum(gamma * (pre - mean) * lax.rsqrt(var + EPS) + beta, 0.0)


def _tc_prep_body(deg2_ref, x_ref, dinv_ref, xs16_ref):
  deg = deg2_ref[0, :N, 0:1] + deg2_ref[1, :N, 0:1] + 1.0
  dinv = lax.rsqrt(deg)
  dinv_ref[...] = dinv
  xs16_ref[...] = jnp.broadcast_to(dinv * x_ref[...], (N, LANES))


def _tc_layer0_body(s0_ref, xs16_ref, dinv_ref, w0_ref, b0_ref, g0_ref,
                    be0_ref, w1_ref, hs_ref):
  dinv = dinv_ref[...]
  t = dinv * (s0_ref[0, :N, 0:1] + s0_ref[1, :N, 0:1] + xs16_ref[:, 0:1])
  pre = t * w0_ref[...] + b0_ref[...]
  h = _bn_relu(pre, g0_ref[...], be0_ref[...])
  hw = jnp.dot(h, w1_ref[...], preferred_element_type=jnp.float32)
  hs = dinv * hw
  hs_ref[0] = hs[:, :HALF]
  hs_ref[1] = hs[:, HALF:]


def _tc_mid_body(acc_ref, hsp_ref, dinv_ref, b_ref, g_ref, be_ref, w_ref,
                 hs_ref):
  dinv = dinv_ref[...]
  h_halves = []
  for c in range(NC):
    pre = dinv * (acc_ref[c, :N] + hsp_ref[c]) + b_ref[:, c * HALF:(c + 1) * HALF]
    h_halves.append(_bn_relu(pre, g_ref[:, c * HALF:(c + 1) * HALF],
                             be_ref[:, c * HALF:(c + 1) * HALF]))
  hw = (jnp.dot(h_halves[0], w_ref[:HALF, :], preferred_element_type=jnp.float32)
        + jnp.dot(h_halves[1], w_ref[HALF:, :], preferred_element_type=jnp.float32))
  hs = dinv * hw
  hs_ref[0] = hs[:, :HALF]
  hs_ref[1] = hs[:, HALF:]


def _tc_final_body(acc_ref, hsp_ref, dinv_ref, b_ref, g_ref, be_ref,
                   l1w_ref, l1b_ref, l2w_ref, l2b_ref, out_ref):
  dinv = dinv_ref[...]
  g_halves = []
  for c in range(NC):
    pre = dinv * (acc_ref[c, :N] + hsp_ref[c]) + b_ref[:, c * HALF:(c + 1) * HALF]
    h = _bn_relu(pre, g_ref[:, c * HALF:(c + 1) * HALF],
                 be_ref[:, c * HALF:(c + 1) * HALF])
    g_halves.append(jnp.mean(h, axis=0, keepdims=True))
  g1 = (jnp.dot(g_halves[0], l1w_ref[:HALF, :], preferred_element_type=jnp.float32)
        + jnp.dot(g_halves[1], l1w_ref[HALF:, :], preferred_element_type=jnp.float32))
  g1 = jnp.maximum(g1 + l1b_ref[...], 0.0)
  out_ref[...] = jnp.dot(g1, l2w_ref[...], preferred_element_type=jnp.float32) + l2b_ref[...]


def _tc(body, out_shapes, *args):
  return pl.pallas_call(
      body, out_shape=out_shapes,
      compiler_params=pltpu.CompilerParams(vmem_limit_bytes=64 * 1024 * 1024),
  )(*args)


def kernel(x, edge_index, params):
  src = edge_index[0].astype(jnp.int32)
  dst = edge_index[1].astype(jnp.int32)
  pad = E_PAD - E
  src_p = jnp.concatenate([src, jnp.zeros((pad,), jnp.int32)])
  dst_p = jnp.concatenate([dst, jnp.full((pad,), N, jnp.int32)])
  src_w = src_p.reshape(NC, NS, CHUNKS_W, CHUNK)
  dst_w = dst_p.reshape(NC, NS, CHUNKS_W, CHUNK)
  src_r = src_p.reshape(NS, CHUNKS2, CH2)
  dst_r = dst_p.reshape(NS, CHUNKS2, CH2)

  convs = params["convs"]
  bns = params["bns"]
  w0 = convs[0]["W"].reshape(1, H)
  bs = [c["b"].reshape(1, H) for c in convs]
  gs = [bn["gamma"].reshape(1, H) for bn in bns]
  bes = [bn["beta"].reshape(1, H) for bn in bns]
  ws = [convs[l]["W"] for l in range(1, 4)]

  deg2 = _sc_hist16(False, jnp.zeros((N, LANES), jnp.float32), src_w, dst_w)
  dinv, xs16 = _tc(
      _tc_prep_body,
      (jax.ShapeDtypeStruct((N, 1), jnp.float32),
       jax.ShapeDtypeStruct((N, LANES), jnp.float32)),
      deg2, x)

  s0 = _sc_hist16(True, xs16, src_w, dst_w)
  hs = _tc(
      _tc_layer0_body,
      jax.ShapeDtypeStruct((NC, N, HALF), jnp.float32),
      s0, xs16, dinv, w0, bs[0], gs[0], bes[0], ws[0])

  for l in (1, 2):
    acc = _sc_rows(hs, src_r, dst_r)
    hs = _tc(
        _tc_mid_body,
        jax.ShapeDtypeStruct((NC, N, HALF), jnp.float32),
        acc, hs, dinv, bs[l], gs[l], bes[l], ws[l])

  acc = _sc_rows(hs, src_r, dst_r)
  out = _tc(
      _tc_final_body,
      jax.ShapeDtypeStruct((1, 1), jnp.float32),
      acc, hs, dinv, bs[3], gs[3], bes[3],
      params["lin1_w"], params["lin1_b"].reshape(1, H),
      params["lin2_w"], params["lin2_b"].reshape(1, 1))
  return out


# speedup vs baseline: 9.3102x; 1.2375x over previous
import functools

import jax
import jax.numpy as jnp
import numpy as np
from jax import lax
from jax.experimental import pallas as pl
from jax.experimental.pallas import tpu as pltpu
from jax.experimental.pallas import tpu_sc as plsc

N = 10000
E = 160000
H = 256
HALF = 128
NC = 2
NS = 16
LANES = 16

CHUNK = 128
E_TOT = E + N
E_PAD = 172032
CH2 = 32
CHUNKS2 = E_PAD // (NS * CH2)
CHUNKS_W = E_PAD // (NC * NS * CHUNK)
ACC_R = 10240
ZSL = ACC_R // NS
RSL = N // NS
RCH = 125
EPS = 1e-5

def _mesh():
  return plsc.VectorSubcoreMesh(core_axis_name="core", subcore_axis_name="subcore")


_SC_PARAMS = pltpu.CompilerParams(use_tc_tiling_on_sc=False,
                                  needs_layout_passes=False)

_PERM = np.empty((H,), np.int32)
for _j in range(H // 32):
  for _k in range(16):
    _PERM[_j * 32 + 2 * _k] = _j * 32 + _k
    _PERM[_j * 32 + 2 * _k + 1] = _j * 32 + 16 + _k


def _fill(ref, nrows, width, value):
  vec = jnp.full((LANES,), value, jnp.float32)

  @pl.loop(0, nrows)
  def _(i):
    for j in range(width // LANES):
      ref[i, pl.ds(j * LANES, LANES)] = vec


def _sc_hist16_body(gather_xs, xs_hbm, src_hbm, dst_hbm, out_hbm,
                    src_v, dst_v, rows_v, stage_v, acc_sh):
  c = lax.axis_index("core")
  s = lax.axis_index("subcore")
  if gather_xs:
    pltpu.sync_copy(src_hbm.at[c, s], src_v)
  pltpu.sync_copy(dst_hbm.at[c, s], dst_v)
  _fill(stage_v, CHUNK, LANES, 0.0)
  @pl.loop(0, ZSL // CHUNK)
  def _(k):
    pltpu.sync_copy(stage_v, acc_sh.at[pl.ds(s * ZSL + k * CHUNK, CHUNK)])
  if not gather_xs:
    _fill(rows_v, CHUNK, LANES, 1.0)
  plsc.subcore_barrier()

  @pl.loop(0, CHUNKS_W)
  def _(j):
    if gather_xs:
      pltpu.sync_copy(xs_hbm.at[src_v.at[j]], rows_v)
    pltpu.sync_copy(rows_v, acc_sh.at[dst_v.at[j]], add=True)

  plsc.subcore_barrier()
  @pl.loop(0, ZSL // CHUNK)
  def _(k):
    base = s * ZSL + k * CHUNK
    pltpu.sync_copy(acc_sh.at[pl.ds(base, CHUNK)], stage_v)
    pltpu.sync_copy(stage_v, out_hbm.at[c].at[pl.ds(base, CHUNK)])


def _sc_rows_body(table_hbm, src_hbm, dst_hbm, out_hbm,
                  src_v, dst_v, gb0, gb1, sb0, sb1, gs0, gs1, ss0, ss1,
                  acc_sh):
  c = lax.axis_index("core")
  s = lax.axis_index("subcore")
  pltpu.sync_copy(src_hbm.at[s], src_v)
  pltpu.sync_copy(dst_hbm.at[s], dst_v)
  _fill(sb0, CH2, HALF, 0.0)
  @pl.loop(0, ZSL // CH2)
  def _(k):
    pltpu.sync_copy(sb0, acc_sh.at[pl.ds(s * ZSL + k * CH2, CH2)])
  plsc.subcore_barrier()

  def g_start(j, buf, sem):
    pltpu.async_copy(table_hbm.at[c].at[src_v.at[j]], buf, sem)

  def g_wait(j, buf, sem):
    pltpu.make_async_copy(table_hbm.at[c].at[src_v.at[j]], buf, sem).wait()

  def s_start(j, buf, sem):
    pltpu.async_copy(buf, acc_sh.at[dst_v.at[j]], sem, add=True)

  def s_wait(j, buf, sem):
    pltpu.make_async_copy(buf, acc_sh.at[dst_v.at[j]], sem).wait()

  def convert(gb, sb):
    @pl.loop(0, CH2)
    def _(i):
      for jb in range(HALF // 32):
        a, b = plsc.unpack(gb[i, pl.ds(jb * 32, 32)],
                           format=plsc.PackFormat.INTERLEAVED)
        sb[i, pl.ds(jb * 32, 16)] = a
        sb[i, pl.ds(jb * 32 + 16, 16)] = b

  g_start(0, gb0, gs0)
  g_start(1, gb1, gs1)
  g_wait(0, gb0, gs0)
  convert(gb0, sb0)
  g_start(2, gb0, gs0)
  s_start(0, sb0, ss0)
  g_wait(1, gb1, gs1)
  convert(gb1, sb1)
  g_start(3, gb1, gs1)
  s_start(1, sb1, ss1)

  @pl.loop(1, CHUNKS2 // 2 - 1)
  def _(k):
    j = 2 * k
    g_wait(j, gb0, gs0)
    s_wait(j - 2, sb0, ss0)
    convert(gb0, sb0)
    g_start(j + 2, gb0, gs0)
    s_start(j, sb0, ss0)
    g_wait(j + 1, gb1, gs1)
    s_wait(j - 1, sb1, ss1)
    convert(gb1, sb1)
    g_start(j + 3, gb1, gs1)
    s_start(j + 1, sb1, ss1)

  jl = CHUNKS2 - 2
  g_wait(jl, gb0, gs0)
  s_wait(jl - 2, sb0, ss0)
  convert(gb0, sb0)
  s_start(jl, sb0, ss0)
  g_wait(jl + 1, gb1, gs1)
  s_wait(jl - 1, sb1, ss1)
  convert(gb1, sb1)
  s_start(jl + 1, sb1, ss1)
  s_wait(jl, sb0, ss0)
  s_wait(jl + 1, sb1, ss1)

  plsc.subcore_barrier()
  @pl.loop(0, ZSL // CH2)
  def _(k):
    base = s * ZSL + k * CH2
    pltpu.sync_copy(acc_sh.at[pl.ds(base, CH2)], sb0)
    pltpu.sync_copy(sb0, out_hbm.at[c].at[pl.ds(base, CH2)])


def _sc_hist16(gather_xs, xs16, src_w, dst_w):
  kern = pl.kernel(
      functools.partial(_sc_hist16_body, gather_xs),
      out_type=jax.ShapeDtypeStruct((NC, ACC_R, LANES), jnp.float32),
      mesh=_mesh(),
      compiler_params=_SC_PARAMS,
      scratch_types=[
          pltpu.VMEM((CHUNKS_W, CHUNK), jnp.int32),
          pltpu.VMEM((CHUNKS_W, CHUNK), jnp.int32),
          pltpu.VMEM((CHUNK, LANES), jnp.float32),
          pltpu.VMEM((CHUNK, LANES), jnp.float32),
          pltpu.VMEM_SHARED((ACC_R, LANES), jnp.float32),
      ],
  )
  return kern(xs16, src_w, dst_w)


def _sc_rows(table, src_r, dst_r):
  kern = pl.kernel(
      _sc_rows_body,
      out_type=jax.ShapeDtypeStruct((NC, ACC_R, HALF), jnp.float32),
      mesh=_mesh(),
      compiler_params=_SC_PARAMS,
      scratch_types=[
          pltpu.VMEM((CHUNKS2, CH2), jnp.int32),
          pltpu.VMEM((CHUNKS2, CH2), jnp.int32),
          pltpu.VMEM((CH2, HALF), jnp.bfloat16),
          pltpu.VMEM((CH2, HALF), jnp.bfloat16),
          pltpu.VMEM((CH2, HALF), jnp.float32),
          pltpu.VMEM((CH2, HALF), jnp.float32),
          pltpu.SemaphoreType.DMA,
          pltpu.SemaphoreType.DMA,
          pltpu.SemaphoreType.DMA,
          pltpu.SemaphoreType.DMA,
          pltpu.VMEM_SHARED((ACC_R, HALF), jnp.float32),
      ],
  )
  return kern(table, src_r, dst_r)


def _dot3(a, b):
  f = jnp.float32
  ahi = a.astype(jnp.bfloat16)
  alo = (a - ahi.astype(f)).astype(jnp.bfloat16)
  bhi = b.astype(jnp.bfloat16)
  blo = (b - bhi.astype(f)).astype(jnp.bfloat16)
  return (jnp.dot(ahi, bhi, preferred_element_type=f)
          + jnp.dot(ahi, blo, preferred_element_type=f)
          + jnp.dot(alo, bhi, preferred_element_type=f))


def _bn_relu(pre, gamma, beta):
  mean = jnp.mean(pre, axis=0, keepdims=True)
  var = jnp.mean((pre - mean) ** 2, axis=0, keepdims=True)
  return jnp.maximum(gamma * (pre - mean) * lax.rsqrt(var + EPS) + beta, 0.0)


def _tc_prep_body(deg2_ref, x_ref, dinv_ref, xs16_ref):
  deg = deg2_ref[0, :N, 0:1] + deg2_ref[1, :N, 0:1]
  dinv = lax.rsqrt(deg)
  dinv_ref[...] = dinv
  xs16_ref[...] = jnp.broadcast_to(dinv * x_ref[...], (N, LANES))


def _split_out(tbl_ref, hsp):
  tbl = hsp.astype(jnp.bfloat16)
  tbl_ref[0] = tbl[:, :HALF]
  tbl_ref[1] = tbl[:, HALF:]


def _tc_layer0_body(s0_ref, dinv_ref, w0_ref, b0_ref, g0_ref,
                    be0_ref, w1p_ref, tbl_ref):
  dinv = dinv_ref[...]
  t = dinv * (s0_ref[0, :N, 0:1] + s0_ref[1, :N, 0:1])
  pre = t * w0_ref[...] + b0_ref[...]
  h = _bn_relu(pre, g0_ref[...], be0_ref[...])
  hsp = dinv * _dot3(h, w1p_ref[...])
  _split_out(tbl_ref, hsp)


def _tc_mid_a_body(acc_ref, dinv_ref, b_ref, g_ref, be_ref, h_ref):
  dinv = dinv_ref[...]
  for c in range(NC):
    pre = dinv * acc_ref[c, :N] + b_ref[:, c * HALF:(c + 1) * HALF]
    h_ref[c] = _bn_relu(pre, g_ref[:, c * HALF:(c + 1) * HALF],
                        be_ref[:, c * HALF:(c + 1) * HALF])


def _tc_mid_b_body(h_ref, dinv_ref, wp_ref, tbl_ref):
  hwp = (_dot3(h_ref[0], wp_ref[:HALF, :])
         + _dot3(h_ref[1], wp_ref[HALF:, :]))
  _split_out(tbl_ref, dinv_ref[...] * hwp)


def _tc_final_body(acc_ref, dinv_ref, b_ref, g_ref, be_ref,
                   l1w_ref, l1b_ref, l2w_ref, l2b_ref, out_ref):
  dinv = dinv_ref[...]
  g_halves = []
  for c in range(NC):
    pre = dinv * acc_ref[c, :N] + b_ref[:, c * HALF:(c + 1) * HALF]
    h = _bn_relu(pre, g_ref[:, c * HALF:(c + 1) * HALF],
                 be_ref[:, c * HALF:(c + 1) * HALF])
    g_halves.append(jnp.mean(h, axis=0, keepdims=True))
  g1 = (_dot3(g_halves[0], l1w_ref[:HALF, :])
        + _dot3(g_halves[1], l1w_ref[HALF:, :]))
  g1 = jnp.maximum(g1 + l1b_ref[...], 0.0)
  out_ref[...] = _dot3(g1, l2w_ref[...]) + l2b_ref[...]


def _tc(body, out_shapes, *args):
  return pl.pallas_call(
      body, out_shape=out_shapes,
      compiler_params=pltpu.CompilerParams(vmem_limit_bytes=64 * 1024 * 1024),
  )(*args)


def kernel(x, edge_index, params):
  src = edge_index[0].astype(jnp.int32)
  dst = edge_index[1].astype(jnp.int32)
  loop = jnp.arange(N, dtype=jnp.int32)
  pad = E_PAD - E_TOT
  src_p = jnp.concatenate([src, loop, jnp.zeros((pad,), jnp.int32)])
  dst_p = jnp.concatenate([dst, loop, jnp.full((pad,), N, jnp.int32)])
  src_w = src_p.reshape(NC, NS, CHUNKS_W, CHUNK)
  dst_w = dst_p.reshape(NC, NS, CHUNKS_W, CHUNK)
  src_r = src_p.reshape(NS, CHUNKS2, CH2)
  dst_r = dst_p.reshape(NS, CHUNKS2, CH2)

  convs = params["convs"]
  bns = params["bns"]
  w0 = convs[0]["W"].reshape(1, H)
  bs = [c["b"].reshape(1, H) for c in convs]
  gs = [bn["gamma"].reshape(1, H) for bn in bns]
  bes = [bn["beta"].reshape(1, H) for bn in bns]
  ws = [convs[l]["W"] for l in range(1, 4)]
  wps = [w[:, _PERM] for w in ws]

  deg2 = _sc_hist16(False, jnp.zeros((N, LANES), jnp.float32), src_w, dst_w)
  dinv, xs16 = _tc(
      _tc_prep_body,
      (jax.ShapeDtypeStruct((N, 1), jnp.float32),
       jax.ShapeDtypeStruct((N, LANES), jnp.float32)),
      deg2, x)

  s0 = _sc_hist16(True, xs16, src_w, dst_w)
  tbl = _tc(
      _tc_layer0_body,
      jax.ShapeDtypeStruct((NC, N, HALF), jnp.bfloat16),
      s0, dinv, w0, bs[0], gs[0], bes[0], wps[0])

  for l in (1, 2):
    acc = _sc_rows(tbl, src_r, dst_r)
    h = _tc(
        _tc_mid_a_body,
        jax.ShapeDtypeStruct((NC, N, HALF), jnp.float32),
        acc, dinv, bs[l], gs[l], bes[l])
    tbl = _tc(
        _tc_mid_b_body,
        jax.ShapeDtypeStruct((NC, N, HALF), jnp.bfloat16),
        h, dinv, wps[l])

  acc = _sc_rows(tbl, src_r, dst_r)
  out = _tc(
      _tc_final_body,
      jax.ShapeDtypeStruct((1, 1), jnp.float32),
      acc, dinv, bs[3], gs[3], bes[3],
      params["lin1_w"], params["lin1_b"].reshape(1, H),
      params["lin2_w"], params["lin2_b"].reshape(1, 1))
  return out


# speedup vs baseline: 9.4972x; 1.0201x over previous
import functools

import jax
import jax.numpy as jnp
import numpy as np
from jax import lax
from jax.experimental import pallas as pl
from jax.experimental.pallas import tpu as pltpu
from jax.experimental.pallas import tpu_sc as plsc

N = 10000
E = 160000
H = 256
HALF = 128
NC = 2
NS = 16
LANES = 16

CHUNK = 128
E_TOT = E + N
E_PAD = 172032
CH2 = 32
CHUNKS2 = E_PAD // (NS * CH2)
CHUNKS_W = E_PAD // (NC * NS * CHUNK)
ACC_R = 10240
ZSL = ACC_R // NS
RSL = N // NS
RCH = 125
EPS = 1e-5

def _mesh():
  return plsc.VectorSubcoreMesh(core_axis_name="core", subcore_axis_name="subcore")


_SC_PARAMS = pltpu.CompilerParams(use_tc_tiling_on_sc=False,
                                  needs_layout_passes=False)

_PERM = np.empty((H,), np.int32)
for _j in range(H // 32):
  for _k in range(16):
    _PERM[_j * 32 + 2 * _k] = _j * 32 + _k
    _PERM[_j * 32 + 2 * _k + 1] = _j * 32 + 16 + _k


def _fill(ref, nrows, width, value):
  vec = jnp.full((LANES,), value, jnp.float32)

  @pl.loop(0, nrows)
  def _(i):
    for j in range(width // LANES):
      ref[i, pl.ds(j * LANES, LANES)] = vec


def _sc_hist16_body(gather_xs, xs_hbm, src_hbm, dst_hbm, out_hbm,
                    src_v, dst_v, rows_v, stage_v, acc_sh):
  c = lax.axis_index("core")
  s = lax.axis_index("subcore")
  if gather_xs:
    pltpu.sync_copy(src_hbm.at[c, s], src_v)
  pltpu.sync_copy(dst_hbm.at[c, s], dst_v)
  _fill(stage_v, CHUNK, LANES, 0.0)
  @pl.loop(0, ZSL // CHUNK)
  def _(k):
    pltpu.sync_copy(stage_v, acc_sh.at[pl.ds(s * ZSL + k * CHUNK, CHUNK)])
  if not gather_xs:
    _fill(rows_v, CHUNK, LANES, 1.0)
  plsc.subcore_barrier()

  @pl.loop(0, CHUNKS_W)
  def _(j):
    if gather_xs:
      pltpu.sync_copy(xs_hbm.at[src_v.at[j]], rows_v)
    pltpu.sync_copy(rows_v, acc_sh.at[dst_v.at[j]], add=True)

  plsc.subcore_barrier()
  @pl.loop(0, ZSL // CHUNK)
  def _(k):
    base = s * ZSL + k * CHUNK
    pltpu.sync_copy(acc_sh.at[pl.ds(base, CHUNK)], stage_v)
    pltpu.sync_copy(stage_v, out_hbm.at[c].at[pl.ds(base, CHUNK)])


def _sc_rows_body(table_hbm, src_hbm, dst_hbm, out_hbm,
                  src_v, dst_v, gb0, gb1, sb0, sb1, gs0, gs1, ss0, ss1,
                  acc_sh):
  c = lax.axis_index("core")
  s = lax.axis_index("subcore")
  pltpu.async_copy(src_hbm.at[s], src_v, gs0)
  pltpu.async_copy(dst_hbm.at[s], dst_v, gs1)
  _fill(sb0, CH2, HALF, 0.0)
  @pl.loop(0, ZSL // CH2)
  def _(k):
    pltpu.async_copy(sb0, acc_sh.at[pl.ds(s * ZSL + k * CH2, CH2)], ss0)
  @pl.loop(0, ZSL // CH2)
  def _(k):
    pltpu.make_async_copy(sb0, acc_sh.at[pl.ds(s * ZSL + k * CH2, CH2)],
                          ss0).wait()
  pltpu.make_async_copy(src_hbm.at[s], src_v, gs0).wait()
  pltpu.make_async_copy(dst_hbm.at[s], dst_v, gs1).wait()
  plsc.subcore_barrier()

  def g_start(j, buf, sem):
    pltpu.async_copy(table_hbm.at[c].at[src_v.at[j]], buf, sem)

  def g_wait(j, buf, sem):
    pltpu.make_async_copy(table_hbm.at[c].at[src_v.at[j]], buf, sem).wait()

  def s_start(j, buf, sem):
    pltpu.async_copy(buf, acc_sh.at[dst_v.at[j]], sem, add=True)

  def s_wait(j, buf, sem):
    pltpu.make_async_copy(buf, acc_sh.at[dst_v.at[j]], sem).wait()

  def convert(gb, sb):
    @pl.loop(0, CH2)
    def _(i):
      for jb in range(HALF // 32):
        a, b = plsc.unpack(gb[i, pl.ds(jb * 32, 32)],
                           format=plsc.PackFormat.INTERLEAVED)
        sb[i, pl.ds(jb * 32, 16)] = a
        sb[i, pl.ds(jb * 32 + 16, 16)] = b

  g_start(0, gb0, gs0)
  g_start(1, gb1, gs1)
  g_wait(0, gb0, gs0)
  convert(gb0, sb0)
  g_start(2, gb0, gs0)
  s_start(0, sb0, ss0)
  g_wait(1, gb1, gs1)
  convert(gb1, sb1)
  g_start(3, gb1, gs1)
  s_start(1, sb1, ss1)

  @pl.loop(1, CHUNKS2 // 2 - 1)
  def _(k):
    j = 2 * k
    g_wait(j, gb0, gs0)
    s_wait(j - 2, sb0, ss0)
    convert(gb0, sb0)
    g_start(j + 2, gb0, gs0)
    s_start(j, sb0, ss0)
    g_wait(j + 1, gb1, gs1)
    s_wait(j - 1, sb1, ss1)
    convert(gb1, sb1)
    g_start(j + 3, gb1, gs1)
    s_start(j + 1, sb1, ss1)

  jl = CHUNKS2 - 2
  g_wait(jl, gb0, gs0)
  s_wait(jl - 2, sb0, ss0)
  convert(gb0, sb0)
  s_start(jl, sb0, ss0)
  g_wait(jl + 1, gb1, gs1)
  s_wait(jl - 1, sb1, ss1)
  convert(gb1, sb1)
  s_start(jl + 1, sb1, ss1)
  s_wait(jl, sb0, ss0)
  s_wait(jl + 1, sb1, ss1)

  plsc.subcore_barrier()
  nrd = ZSL // CH2
  sbs, wsems = (sb0, sb1), (ss0, ss1)

  def rd(k, b):
    base = s * ZSL + k * CH2
    pltpu.sync_copy(acc_sh.at[pl.ds(base, CH2)], sbs[b])
    pltpu.async_copy(sbs[b], out_hbm.at[c].at[pl.ds(base, CH2)], wsems[b])

  def rd_wait(k, b):
    base = s * ZSL + k * CH2
    pltpu.make_async_copy(sbs[b], out_hbm.at[c].at[pl.ds(base, CH2)],
                          wsems[b]).wait()

  rd(0, 0)
  rd(1, 1)
  @pl.loop(1, nrd // 2)
  def _(k):
    rd_wait(2 * k - 2, 0)
    rd(2 * k, 0)
    rd_wait(2 * k - 1, 1)
    rd(2 * k + 1, 1)
  rd_wait(nrd - 2, 0)
  rd_wait(nrd - 1, 1)


def _sc_hist16(gather_xs, xs16, src_w, dst_w):
  kern = pl.kernel(
      functools.partial(_sc_hist16_body, gather_xs),
      out_type=jax.ShapeDtypeStruct((NC, ACC_R, LANES), jnp.float32),
      mesh=_mesh(),
      compiler_params=_SC_PARAMS,
      scratch_types=[
          pltpu.VMEM((CHUNKS_W, CHUNK), jnp.int32),
          pltpu.VMEM((CHUNKS_W, CHUNK), jnp.int32),
          pltpu.VMEM((CHUNK, LANES), jnp.float32),
          pltpu.VMEM((CHUNK, LANES), jnp.float32),
          pltpu.VMEM_SHARED((ACC_R, LANES), jnp.float32),
      ],
  )
  return kern(xs16, src_w, dst_w)


def _sc_rows(table, src_r, dst_r):
  kern = pl.kernel(
      _sc_rows_body,
      out_type=jax.ShapeDtypeStruct((NC, ACC_R, HALF), jnp.float32),
      mesh=_mesh(),
      compiler_params=_SC_PARAMS,
      scratch_types=[
          pltpu.VMEM((CHUNKS2, CH2), jnp.int32),
          pltpu.VMEM((CHUNKS2, CH2), jnp.int32),
          pltpu.VMEM((CH2, HALF), jnp.bfloat16),
          pltpu.VMEM((CH2, HALF), jnp.bfloat16),
          pltpu.VMEM((CH2, HALF), jnp.float32),
          pltpu.VMEM((CH2, HALF), jnp.float32),
          pltpu.SemaphoreType.DMA,
          pltpu.SemaphoreType.DMA,
          pltpu.SemaphoreType.DMA,
          pltpu.SemaphoreType.DMA,
          pltpu.VMEM_SHARED((ACC_R, HALF), jnp.float32),
      ],
  )
  return kern(table, src_r, dst_r)


def _dot3(a, b):
  f = jnp.float32
  ahi = a.astype(jnp.bfloat16)
  alo = (a - ahi.astype(f)).astype(jnp.bfloat16)
  bhi = b.astype(jnp.bfloat16)
  blo = (b - bhi.astype(f)).astype(jnp.bfloat16)
  return (jnp.dot(ahi, bhi, preferred_element_type=f)
          + jnp.dot(ahi, blo, preferred_element_type=f)
          + jnp.dot(alo, bhi, preferred_element_type=f))


def _bn_relu(pre, gamma, beta):
  mean = jnp.mean(pre, axis=0, keepdims=True)
  var = jnp.mean((pre - mean) ** 2, axis=0, keepdims=True)
  return jnp.maximum(gamma * (pre - mean) * lax.rsqrt(var + EPS) + beta, 0.0)


def _tc_prep_body(deg2_ref, x_ref, dinv_ref, xs16_ref):
  deg = deg2_ref[0, :N, 0:1] + deg2_ref[1, :N, 0:1]
  dinv = lax.rsqrt(deg)
  dinv_ref[...] = dinv
  xs16_ref[...] = jnp.broadcast_to(dinv * x_ref[...], (N, LANES))


def _split_out(tbl_ref, hsp):
  tbl = hsp.astype(jnp.bfloat16)
  tbl_ref[0] = tbl[:, :HALF]
  tbl_ref[1] = tbl[:, HALF:]


def _tc_layer0_body(s0_ref, dinv_ref, w0_ref, b0_ref, g0_ref,
                    be0_ref, w1p_ref, tbl_ref):
  dinv = dinv_ref[...]
  t = dinv * (s0_ref[0, :N, 0:1] + s0_ref[1, :N, 0:1])
  pre = t * w0_ref[...] + b0_ref[...]
  h = _bn_relu(pre, g0_ref[...], be0_ref[...])
  hsp = dinv * _dot3(h, w1p_ref[...])
  _split_out(tbl_ref, hsp)


def _tc_mid_a_body(acc_ref, dinv_ref, b_ref, g_ref, be_ref, h_ref):
  dinv = dinv_ref[...]
  for c in range(NC):
    pre = dinv * acc_ref[c, :N] + b_ref[:, c * HALF:(c + 1) * HALF]
    h_ref[c] = _bn_relu(pre, g_ref[:, c * HALF:(c + 1) * HALF],
                        be_ref[:, c * HALF:(c + 1) * HALF])


def _tc_mid_b_body(h_ref, dinv_ref, wp_ref, tbl_ref):
  hwp = (_dot3(h_ref[0], wp_ref[:HALF, :])
         + _dot3(h_ref[1], wp_ref[HALF:, :]))
  _split_out(tbl_ref, dinv_ref[...] * hwp)


def _tc_final_body(acc_ref, dinv_ref, b_ref, g_ref, be_ref,
                   l1w_ref, l1b_ref, l2w_ref, l2b_ref, out_ref):
  dinv = dinv_ref[...]
  g_halves = []
  for c in range(NC):
    pre = dinv * acc_ref[c, :N] + b_ref[:, c * HALF:(c + 1) * HALF]
    h = _bn_relu(pre, g_ref[:, c * HALF:(c + 1) * HALF],
                 be_ref[:, c * HALF:(c + 1) * HALF])
    g_halves.append(jnp.mean(h, axis=0, keepdims=True))
  g1 = (_dot3(g_halves[0], l1w_ref[:HALF, :])
        + _dot3(g_halves[1], l1w_ref[HALF:, :]))
  g1 = jnp.maximum(g1 + l1b_ref[...], 0.0)
  out_ref[...] = _dot3(g1, l2w_ref[...]) + l2b_ref[...]


def _tc(body, out_shapes, *args):
  return pl.pallas_call(
      body, out_shape=out_shapes,
      compiler_params=pltpu.CompilerParams(vmem_limit_bytes=64 * 1024 * 1024),
  )(*args)


def kernel(x, edge_index, params):
  src = edge_index[0].astype(jnp.int32)
  dst = edge_index[1].astype(jnp.int32)
  loop = jnp.arange(N, dtype=jnp.int32)
  pad = E_PAD - E_TOT
  src_p = jnp.concatenate([src, loop, jnp.zeros((pad,), jnp.int32)])
  dst_p = jnp.concatenate([dst, loop, jnp.full((pad,), N, jnp.int32)])
  src_w = src_p.reshape(NC, NS, CHUNKS_W, CHUNK)
  dst_w = dst_p.reshape(NC, NS, CHUNKS_W, CHUNK)
  src_r = src_p.reshape(NS, CHUNKS2, CH2)
  dst_r = dst_p.reshape(NS, CHUNKS2, CH2)

  convs = params["convs"]
  bns = params["bns"]
  w0 = convs[0]["W"].reshape(1, H)
  bs = [c["b"].reshape(1, H) for c in convs]
  gs = [bn["gamma"].reshape(1, H) for bn in bns]
  bes = [bn["beta"].reshape(1, H) for bn in bns]
  ws = [convs[l]["W"] for l in range(1, 4)]
  wps = [w[:, _PERM] for w in ws]

  deg2 = _sc_hist16(False, jnp.zeros((N, LANES), jnp.float32), src_w, dst_w)
  dinv, xs16 = _tc(
      _tc_prep_body,
      (jax.ShapeDtypeStruct((N, 1), jnp.float32),
       jax.ShapeDtypeStruct((N, LANES), jnp.float32)),
      deg2, x)

  s0 = _sc_hist16(True, xs16, src_w, dst_w)
  tbl = _tc(
      _tc_layer0_body,
      jax.ShapeDtypeStruct((NC, N, HALF), jnp.bfloat16),
      s0, dinv, w0, bs[0], gs[0], bes[0], wps[0])

  for l in (1, 2):
    acc = _sc_rows(tbl, src_r, dst_r)
    h = _tc(
        _tc_mid_a_body,
        jax.ShapeDtypeStruct((NC, N, HALF), jnp.float32),
        acc, dinv, bs[l], gs[l], bes[l])
    tbl = _tc(
        _tc_mid_b_body,
        jax.ShapeDtypeStruct((NC, N, HALF), jnp.bfloat16),
        h, dinv, wps[l])

  acc = _sc_rows(tbl, src_r, dst_r)
  out = _tc(
      _tc_final_body,
      jax.ShapeDtypeStruct((1, 1), jnp.float32),
      acc, dinv, bs[3], gs[3], bes[3],
      params["lin1_w"], params["lin1_b"].reshape(1, H),
      params["lin2_w"], params["lin2_b"].reshape(1, 1))
  return out


# speedup vs baseline: 9.8926x; 1.0416x over previous
import functools

import jax
import jax.numpy as jnp
import numpy as np
from jax import lax
from jax.experimental import pallas as pl
from jax.experimental.pallas import tpu as pltpu
from jax.experimental.pallas import tpu_sc as plsc

N = 10000
E = 160000
H = 256
HALF = 128
NC = 2
NS = 16
LANES = 16

CHUNK = 128
E_TOT = E + N
E_PAD = 172032
CH2 = 48
CHUNKS2 = E_PAD // (NS * CH2)
RZ = 32
CHUNKS_W = E_PAD // (NC * NS * CHUNK)
ACC_R = 10240
ZSL = ACC_R // NS
RSL = N // NS
RCH = 125
EPS = 1e-5

def _mesh():
  return plsc.VectorSubcoreMesh(core_axis_name="core", subcore_axis_name="subcore")


_SC_PARAMS = pltpu.CompilerParams(use_tc_tiling_on_sc=False,
                                  needs_layout_passes=False)

_PERM = np.empty((H,), np.int32)
for _j in range(H // 32):
  for _k in range(16):
    _PERM[_j * 32 + 2 * _k] = _j * 32 + _k
    _PERM[_j * 32 + 2 * _k + 1] = _j * 32 + 16 + _k


def _fill(ref, nrows, width, value):
  vec = jnp.full((LANES,), value, jnp.float32)

  @pl.loop(0, nrows)
  def _(i):
    for j in range(width // LANES):
      ref[i, pl.ds(j * LANES, LANES)] = vec


def _sc_hist16_body(gather_xs, xs_hbm, src_hbm, dst_hbm, out_hbm,
                    src_v, dst_v, rows_v, stage_v, acc_sh):
  c = lax.axis_index("core")
  s = lax.axis_index("subcore")
  if gather_xs:
    pltpu.sync_copy(src_hbm.at[c, s], src_v)
  pltpu.sync_copy(dst_hbm.at[c, s], dst_v)
  _fill(stage_v, CHUNK, LANES, 0.0)
  @pl.loop(0, ZSL // CHUNK)
  def _(k):
    pltpu.sync_copy(stage_v, acc_sh.at[pl.ds(s * ZSL + k * CHUNK, CHUNK)])
  if not gather_xs:
    _fill(rows_v, CHUNK, LANES, 1.0)
  plsc.subcore_barrier()

  @pl.loop(0, CHUNKS_W)
  def _(j):
    if gather_xs:
      pltpu.sync_copy(xs_hbm.at[src_v.at[j]], rows_v)
    pltpu.sync_copy(rows_v, acc_sh.at[dst_v.at[j]], add=True)

  plsc.subcore_barrier()
  @pl.loop(0, ZSL // CHUNK)
  def _(k):
    base = s * ZSL + k * CHUNK
    pltpu.sync_copy(acc_sh.at[pl.ds(base, CHUNK)], stage_v)
    pltpu.sync_copy(stage_v, out_hbm.at[c].at[pl.ds(base, CHUNK)])


def _sc_rows_body(table_hbm, src_hbm, dst_hbm, out_hbm,
                  src_v, dst_v, gb0, gb1, sb0, sb1, gs0, gs1, ss0, ss1,
                  acc_sh):
  c = lax.axis_index("core")
  s = lax.axis_index("subcore")
  pltpu.async_copy(src_hbm.at[s], src_v, gs0)
  pltpu.async_copy(dst_hbm.at[s], dst_v, gs1)
  _fill(sb0, RZ, HALF, 0.0)
  @pl.loop(0, ZSL // RZ)
  def _(k):
    pltpu.async_copy(sb0.at[pl.ds(0, RZ)],
                     acc_sh.at[pl.ds(s * ZSL + k * RZ, RZ)], ss0)
  @pl.loop(0, ZSL // RZ)
  def _(k):
    pltpu.make_async_copy(sb0.at[pl.ds(0, RZ)],
                          acc_sh.at[pl.ds(s * ZSL + k * RZ, RZ)],
                          ss0).wait()
  pltpu.make_async_copy(src_hbm.at[s], src_v, gs0).wait()
  pltpu.make_async_copy(dst_hbm.at[s], dst_v, gs1).wait()
  plsc.subcore_barrier()

  def g_start(j, buf, sem):
    pltpu.async_copy(table_hbm.at[c].at[src_v.at[j]], buf, sem)

  def g_wait(j, buf, sem):
    pltpu.make_async_copy(table_hbm.at[c].at[src_v.at[j]], buf, sem).wait()

  def s_start(j, buf, sem):
    pltpu.async_copy(buf, acc_sh.at[dst_v.at[j]], sem, add=True)

  def s_wait(j, buf, sem):
    pltpu.make_async_copy(buf, acc_sh.at[dst_v.at[j]], sem).wait()

  def convert(gb, sb):
    @pl.loop(0, CH2)
    def _(i):
      for jb in range(HALF // 32):
        a, b = plsc.unpack(gb[i, pl.ds(jb * 32, 32)],
                           format=plsc.PackFormat.INTERLEAVED)
        sb[i, pl.ds(jb * 32, 16)] = a
        sb[i, pl.ds(jb * 32 + 16, 16)] = b

  g_start(0, gb0, gs0)
  g_start(1, gb1, gs1)
  g_wait(0, gb0, gs0)
  convert(gb0, sb0)
  g_start(2, gb0, gs0)
  s_start(0, sb0, ss0)
  g_wait(1, gb1, gs1)
  convert(gb1, sb1)
  g_start(3, gb1, gs1)
  s_start(1, sb1, ss1)

  @pl.loop(1, CHUNKS2 // 2 - 1)
  def _(k):
    j = 2 * k
    g_wait(j, gb0, gs0)
    s_wait(j - 2, sb0, ss0)
    convert(gb0, sb0)
    g_start(j + 2, gb0, gs0)
    s_start(j, sb0, ss0)
    g_wait(j + 1, gb1, gs1)
    s_wait(j - 1, sb1, ss1)
    convert(gb1, sb1)
    g_start(j + 3, gb1, gs1)
    s_start(j + 1, sb1, ss1)

  jl = CHUNKS2 - 2
  g_wait(jl, gb0, gs0)
  s_wait(jl - 2, sb0, ss0)
  convert(gb0, sb0)
  s_start(jl, sb0, ss0)
  g_wait(jl + 1, gb1, gs1)
  s_wait(jl - 1, sb1, ss1)
  convert(gb1, sb1)
  s_start(jl + 1, sb1, ss1)
  s_wait(jl, sb0, ss0)
  s_wait(jl + 1, sb1, ss1)

  plsc.subcore_barrier()
  nrd = ZSL // RZ
  sbs, wsems = (sb0, sb1), (ss0, ss1)

  def rd(k, b):
    base = s * ZSL + k * RZ
    pltpu.sync_copy(acc_sh.at[pl.ds(base, RZ)], sbs[b].at[pl.ds(0, RZ)])
    pltpu.async_copy(sbs[b].at[pl.ds(0, RZ)],
                     out_hbm.at[c].at[pl.ds(base, RZ)], wsems[b])

  def rd_wait(k, b):
    base = s * ZSL + k * RZ
    pltpu.make_async_copy(sbs[b].at[pl.ds(0, RZ)],
                          out_hbm.at[c].at[pl.ds(base, RZ)],
                          wsems[b]).wait()

  rd(0, 0)
  rd(1, 1)
  @pl.loop(1, nrd // 2)
  def _(k):
    rd_wait(2 * k - 2, 0)
    rd(2 * k, 0)
    rd_wait(2 * k - 1, 1)
    rd(2 * k + 1, 1)
  rd_wait(nrd - 2, 0)
  rd_wait(nrd - 1, 1)


def _sc_hist16(gather_xs, xs16, src_w, dst_w):
  kern = pl.kernel(
      functools.partial(_sc_hist16_body, gather_xs),
      out_type=jax.ShapeDtypeStruct((NC, ACC_R, LANES), jnp.float32),
      mesh=_mesh(),
      compiler_params=_SC_PARAMS,
      scratch_types=[
          pltpu.VMEM((CHUNKS_W, CHUNK), jnp.int32),
          pltpu.VMEM((CHUNKS_W, CHUNK), jnp.int32),
          pltpu.VMEM((CHUNK, LANES), jnp.float32),
          pltpu.VMEM((CHUNK, LANES), jnp.float32),
          pltpu.VMEM_SHARED((ACC_R, LANES), jnp.float32),
      ],
  )
  return kern(xs16, src_w, dst_w)


def _sc_rows(table, src_r, dst_r):
  kern = pl.kernel(
      _sc_rows_body,
      out_type=jax.ShapeDtypeStruct((NC, ACC_R, HALF), jnp.float32),
      mesh=_mesh(),
      compiler_params=_SC_PARAMS,
      scratch_types=[
          pltpu.VMEM((CHUNKS2, CH2), jnp.int32),
          pltpu.VMEM((CHUNKS2, CH2), jnp.int32),
          pltpu.VMEM((CH2, HALF), jnp.bfloat16),
          pltpu.VMEM((CH2, HALF), jnp.bfloat16),
          pltpu.VMEM((CH2, HALF), jnp.float32),
          pltpu.VMEM((CH2, HALF), jnp.float32),
          pltpu.SemaphoreType.DMA,
          pltpu.SemaphoreType.DMA,
          pltpu.SemaphoreType.DMA,
          pltpu.SemaphoreType.DMA,
          pltpu.VMEM_SHARED((ACC_R, HALF), jnp.float32),
      ],
  )
  return kern(table, src_r, dst_r)


def _dot3(a, b):
  f = jnp.float32
  ahi = a.astype(jnp.bfloat16)
  alo = (a - ahi.astype(f)).astype(jnp.bfloat16)
  bhi = b.astype(jnp.bfloat16)
  blo = (b - bhi.astype(f)).astype(jnp.bfloat16)
  return (jnp.dot(ahi, bhi, preferred_element_type=f)
          + jnp.dot(ahi, blo, preferred_element_type=f)
          + jnp.dot(alo, bhi, preferred_element_type=f))


def _bn_relu(pre, gamma, beta):
  mean = jnp.mean(pre, axis=0, keepdims=True)
  var = jnp.mean((pre - mean) ** 2, axis=0, keepdims=True)
  return jnp.maximum(gamma * (pre - mean) * lax.rsqrt(var + EPS) + beta, 0.0)


def _tc_prep_body(deg2_ref, x_ref, dinv_ref, xs16_ref):
  deg = deg2_ref[0, :N, 0:1] + deg2_ref[1, :N, 0:1]
  dinv = lax.rsqrt(deg)
  dinv_ref[...] = dinv
  xs16_ref[...] = jnp.broadcast_to(dinv * x_ref[...], (N, LANES))


def _split_out(tbl_ref, hsp):
  tbl = hsp.astype(jnp.bfloat16)
  tbl_ref[0] = tbl[:, :HALF]
  tbl_ref[1] = tbl[:, HALF:]


def _tc_layer0_body(s0_ref, dinv_ref, w0_ref, b0_ref, g0_ref,
                    be0_ref, w1p_ref, tbl_ref):
  dinv = dinv_ref[...]
  t = dinv * (s0_ref[0, :N, 0:1] + s0_ref[1, :N, 0:1])
  pre = t * w0_ref[...] + b0_ref[...]
  h = _bn_relu(pre, g0_ref[...], be0_ref[...])
  hsp = dinv * _dot3(h, w1p_ref[...])
  _split_out(tbl_ref, hsp)


def _tc_mid_a_body(acc_ref, dinv_ref, b_ref, g_ref, be_ref, h_ref):
  dinv = dinv_ref[...]
  for c in range(NC):
    pre = dinv * acc_ref[c, :N] + b_ref[:, c * HALF:(c + 1) * HALF]
    h_ref[c] = _bn_relu(pre, g_ref[:, c * HALF:(c + 1) * HALF],
                        be_ref[:, c * HALF:(c + 1) * HALF])


def _tc_mid_b_body(h_ref, dinv_ref, wp_ref, tbl_ref):
  hwp = (_dot3(h_ref[0], wp_ref[:HALF, :])
         + _dot3(h_ref[1], wp_ref[HALF:, :]))
  _split_out(tbl_ref, dinv_ref[...] * hwp)


def _tc_final_body(acc_ref, dinv_ref, b_ref, g_ref, be_ref,
                   l1w_ref, l1b_ref, l2w_ref, l2b_ref, out_ref):
  dinv = dinv_ref[...]
  g_halves = []
  for c in range(NC):
    pre = dinv * acc_ref[c, :N] + b_ref[:, c * HALF:(c + 1) * HALF]
    h = _bn_relu(pre, g_ref[:, c * HALF:(c + 1) * HALF],
                 be_ref[:, c * HALF:(c + 1) * HALF])
    g_halves.append(jnp.mean(h, axis=0, keepdims=True))
  g1 = (_dot3(g_halves[0], l1w_ref[:HALF, :])
        + _dot3(g_halves[1], l1w_ref[HALF:, :]))
  g1 = jnp.maximum(g1 + l1b_ref[...], 0.0)
  out_ref[...] = _dot3(g1, l2w_ref[...]) + l2b_ref[...]


def _tc(body, out_shapes, *args):
  return pl.pallas_call(
      body, out_shape=out_shapes,
      compiler_params=pltpu.CompilerParams(vmem_limit_bytes=64 * 1024 * 1024),
  )(*args)


def kernel(x, edge_index, params):
  src = edge_index[0].astype(jnp.int32)
  dst = edge_index[1].astype(jnp.int32)
  loop = jnp.arange(N, dtype=jnp.int32)
  pad = E_PAD - E_TOT
  src_p = jnp.concatenate([src, loop, jnp.zeros((pad,), jnp.int32)])
  dst_p = jnp.concatenate([dst, loop, jnp.full((pad,), N, jnp.int32)])
  src_w = src_p.reshape(NC, NS, CHUNKS_W, CHUNK)
  dst_w = dst_p.reshape(NC, NS, CHUNKS_W, CHUNK)
  src_r = src_p.reshape(NS, CHUNKS2, CH2)
  dst_r = dst_p.reshape(NS, CHUNKS2, CH2)

  convs = params["convs"]
  bns = params["bns"]
  w0 = convs[0]["W"].reshape(1, H)
  bs = [c["b"].reshape(1, H) for c in convs]
  gs = [bn["gamma"].reshape(1, H) for bn in bns]
  bes = [bn["beta"].reshape(1, H) for bn in bns]
  ws = [convs[l]["W"] for l in range(1, 4)]
  wps = [w[:, _PERM] for w in ws]

  deg2 = _sc_hist16(False, jnp.zeros((N, LANES), jnp.float32), src_w, dst_w)
  dinv, xs16 = _tc(
      _tc_prep_body,
      (jax.ShapeDtypeStruct((N, 1), jnp.float32),
       jax.ShapeDtypeStruct((N, LANES), jnp.float32)),
      deg2, x)

  s0 = _sc_hist16(True, xs16, src_w, dst_w)
  tbl = _tc(
      _tc_layer0_body,
      jax.ShapeDtypeStruct((NC, N, HALF), jnp.bfloat16),
      s0, dinv, w0, bs[0], gs[0], bes[0], wps[0])

  for l in (1, 2):
    acc = _sc_rows(tbl, src_r, dst_r)
    h = _tc(
        _tc_mid_a_body,
        jax.ShapeDtypeStruct((NC, N, HALF), jnp.float32),
        acc, dinv, bs[l], gs[l], bes[l])
    tbl = _tc(
        _tc_mid_b_body,
        jax.ShapeDtypeStruct((NC, N, HALF), jnp.bfloat16),
        h, dinv, wps[l])

  acc = _sc_rows(tbl, src_r, dst_r)
  out = _tc(
      _tc_final_body,
      jax.ShapeDtypeStruct((1, 1), jnp.float32),
      acc, dinv, bs[3], gs[3], bes[3],
      params["lin1_w"], params["lin1_b"].reshape(1, H),
      params["lin2_w"], params["lin2_b"].reshape(1, 1))
  return out


# speedup vs baseline: 10.0482x; 1.0157x over previous
import functools

import jax
import jax.numpy as jnp
import numpy as np
from jax import lax
from jax.experimental import pallas as pl
from jax.experimental.pallas import tpu as pltpu
from jax.experimental.pallas import tpu_sc as plsc

N = 10000
E = 160000
H = 256
HALF = 128
NC = 2
NS = 16
LANES = 16

CHUNK = 128
E_TOT = E + N
E_PAD = 172032
CH2 = 48
CHUNKS2 = E_PAD // (NS * CH2)
RZ = 32
CHUNKS_W = E_PAD // (NC * NS * CHUNK)
ACC_R = 10240
ZSL = ACC_R // NS
RSL = N // NS
RCH = 125
EPS = 1e-5

def _mesh():
  return plsc.VectorSubcoreMesh(core_axis_name="core", subcore_axis_name="subcore")


_SC_PARAMS = pltpu.CompilerParams(use_tc_tiling_on_sc=False,
                                  needs_layout_passes=False)

_PERM = np.empty((H,), np.int32)
for _j in range(H // 32):
  for _k in range(16):
    _PERM[_j * 32 + 2 * _k] = _j * 32 + _k
    _PERM[_j * 32 + 2 * _k + 1] = _j * 32 + 16 + _k


def _fill(ref, nrows, width, value):
  vec = jnp.full((LANES,), value, jnp.float32)

  @pl.loop(0, nrows)
  def _(i):
    for j in range(width // LANES):
      ref[i, pl.ds(j * LANES, LANES)] = vec


def _sc_hist16_body(gather_xs, xs_hbm, src_hbm, dst_hbm, out_hbm,
                    src_v, dst_v, rows_v, stage_v, hg0, hg1, hs0, hs1,
                    acc_sh):
  c = lax.axis_index("core")
  s = lax.axis_index("subcore")
  if gather_xs:
    pltpu.sync_copy(src_hbm.at[c, s], src_v)
  pltpu.sync_copy(dst_hbm.at[c, s], dst_v)
  _fill(stage_v, CHUNK, LANES, 0.0)
  @pl.loop(0, ZSL // CHUNK)
  def _(k):
    pltpu.sync_copy(stage_v, acc_sh.at[pl.ds(s * ZSL + k * CHUNK, CHUNK)])
  if not gather_xs:
    _fill(rows_v, CHUNK, LANES, 1.0)
  plsc.subcore_barrier()

  if gather_xs:
    def g_start(j, buf, sem):
      pltpu.async_copy(xs_hbm.at[src_v.at[j]], buf, sem)

    def g_wait(j, buf, sem):
      pltpu.make_async_copy(xs_hbm.at[src_v.at[j]], buf, sem).wait()

    def s_start(j, buf, sem):
      pltpu.async_copy(buf, acc_sh.at[dst_v.at[j]], sem, add=True)

    def s_wait(j, buf, sem):
      pltpu.make_async_copy(buf, acc_sh.at[dst_v.at[j]], sem).wait()

    bufs = (rows_v, stage_v)
    gsems = (hg0, hg1)
    ssems = (hs0, hs1)
    g_start(0, bufs[0], gsems[0])
    g_wait(0, bufs[0], gsems[0])
    g_start(1, bufs[1], gsems[1])
    s_start(0, bufs[0], ssems[0])
    g_wait(1, bufs[1], gsems[1])
    s_wait(0, bufs[0], ssems[0])
    g_start(2, bufs[0], gsems[0])
    s_start(1, bufs[1], ssems[1])

    @pl.loop(1, CHUNKS_W // 2 - 1)
    def _(k):
      j = 2 * k
      g_wait(j, bufs[0], gsems[0])
      s_wait(j - 1, bufs[1], ssems[1])
      g_start(j + 1, bufs[1], gsems[1])
      s_start(j, bufs[0], ssems[0])
      g_wait(j + 1, bufs[1], gsems[1])
      s_wait(j, bufs[0], ssems[0])
      g_start(j + 2, bufs[0], gsems[0])
      s_start(j + 1, bufs[1], ssems[1])

    jl = CHUNKS_W - 2
    g_wait(jl, bufs[0], gsems[0])
    s_wait(jl - 1, bufs[1], ssems[1])
    g_start(jl + 1, bufs[1], gsems[1])
    s_start(jl, bufs[0], ssems[0])
    g_wait(jl + 1, bufs[1], gsems[1])
    s_start(jl + 1, bufs[1], ssems[1])
    s_wait(jl, bufs[0], ssems[0])
    s_wait(jl + 1, bufs[1], ssems[1])
  else:
    @pl.loop(0, CHUNKS_W)
    def _(j):
      pltpu.sync_copy(rows_v, acc_sh.at[dst_v.at[j]], add=True)

  plsc.subcore_barrier()
  @pl.loop(0, ZSL // CHUNK)
  def _(k):
    base = s * ZSL + k * CHUNK
    pltpu.sync_copy(acc_sh.at[pl.ds(base, CHUNK)], stage_v)
    pltpu.sync_copy(stage_v, out_hbm.at[c].at[pl.ds(base, CHUNK)])


def _sc_rows_body(table_hbm, src_hbm, dst_hbm, out_hbm,
                  src_v, dst_v, gb0, gb1, sb0, sb1, gs0, gs1, ss0, ss1,
                  acc_sh):
  c = lax.axis_index("core")
  s = lax.axis_index("subcore")
  pltpu.async_copy(src_hbm.at[s], src_v, gs0)
  pltpu.async_copy(dst_hbm.at[s], dst_v, gs1)
  _fill(sb0, RZ, HALF, 0.0)
  @pl.loop(0, ZSL // RZ)
  def _(k):
    pltpu.async_copy(sb0.at[pl.ds(0, RZ)],
                     acc_sh.at[pl.ds(s * ZSL + k * RZ, RZ)], ss0)
  @pl.loop(0, ZSL // RZ)
  def _(k):
    pltpu.make_async_copy(sb0.at[pl.ds(0, RZ)],
                          acc_sh.at[pl.ds(s * ZSL + k * RZ, RZ)],
                          ss0).wait()
  pltpu.make_async_copy(src_hbm.at[s], src_v, gs0).wait()
  pltpu.make_async_copy(dst_hbm.at[s], dst_v, gs1).wait()
  plsc.subcore_barrier()

  def g_start(j, buf, sem):
    pltpu.async_copy(table_hbm.at[c].at[src_v.at[j]], buf, sem)

  def g_wait(j, buf, sem):
    pltpu.make_async_copy(table_hbm.at[c].at[src_v.at[j]], buf, sem).wait()

  def s_start(j, buf, sem):
    pltpu.async_copy(buf, acc_sh.at[dst_v.at[j]], sem, add=True)

  def s_wait(j, buf, sem):
    pltpu.make_async_copy(buf, acc_sh.at[dst_v.at[j]], sem).wait()

  def convert(gb, sb):
    @pl.loop(0, CH2)
    def _(i):
      for jb in range(HALF // 32):
        a, b = plsc.unpack(gb[i, pl.ds(jb * 32, 32)],
                           format=plsc.PackFormat.INTERLEAVED)
        sb[i, pl.ds(jb * 32, 16)] = a
        sb[i, pl.ds(jb * 32 + 16, 16)] = b

  g_start(0, gb0, gs0)
  g_start(1, gb1, gs1)
  g_wait(0, gb0, gs0)
  convert(gb0, sb0)
  g_start(2, gb0, gs0)
  s_start(0, sb0, ss0)
  g_wait(1, gb1, gs1)
  convert(gb1, sb1)
  g_start(3, gb1, gs1)
  s_start(1, sb1, ss1)

  @pl.loop(1, CHUNKS2 // 2 - 1)
  def _(k):
    j = 2 * k
    g_wait(j, gb0, gs0)
    s_wait(j - 2, sb0, ss0)
    convert(gb0, sb0)
    g_start(j + 2, gb0, gs0)
    s_start(j, sb0, ss0)
    g_wait(j + 1, gb1, gs1)
    s_wait(j - 1, sb1, ss1)
    convert(gb1, sb1)
    g_start(j + 3, gb1, gs1)
    s_start(j + 1, sb1, ss1)

  jl = CHUNKS2 - 2
  g_wait(jl, gb0, gs0)
  s_wait(jl - 2, sb0, ss0)
  convert(gb0, sb0)
  s_start(jl, sb0, ss0)
  g_wait(jl + 1, gb1, gs1)
  s_wait(jl - 1, sb1, ss1)
  convert(gb1, sb1)
  s_start(jl + 1, sb1, ss1)
  s_wait(jl, sb0, ss0)
  s_wait(jl + 1, sb1, ss1)

  plsc.subcore_barrier()
  nrd = ZSL // RZ
  sbs, wsems = (sb0, sb1), (ss0, ss1)

  def rd(k, b):
    base = s * ZSL + k * RZ
    pltpu.sync_copy(acc_sh.at[pl.ds(base, RZ)], sbs[b].at[pl.ds(0, RZ)])
    pltpu.async_copy(sbs[b].at[pl.ds(0, RZ)],
                     out_hbm.at[c].at[pl.ds(base, RZ)], wsems[b])

  def rd_wait(k, b):
    base = s * ZSL + k * RZ
    pltpu.make_async_copy(sbs[b].at[pl.ds(0, RZ)],
                          out_hbm.at[c].at[pl.ds(base, RZ)],
                          wsems[b]).wait()

  rd(0, 0)
  rd(1, 1)
  @pl.loop(1, nrd // 2)
  def _(k):
    rd_wait(2 * k - 2, 0)
    rd(2 * k, 0)
    rd_wait(2 * k - 1, 1)
    rd(2 * k + 1, 1)
  rd_wait(nrd - 2, 0)
  rd_wait(nrd - 1, 1)


def _sc_hist16(gather_xs, xs16, src_w, dst_w):
  kern = pl.kernel(
      functools.partial(_sc_hist16_body, gather_xs),
      out_type=jax.ShapeDtypeStruct((NC, ACC_R, LANES), jnp.float32),
      mesh=_mesh(),
      compiler_params=_SC_PARAMS,
      scratch_types=[
          pltpu.VMEM((CHUNKS_W, CHUNK), jnp.int32),
          pltpu.VMEM((CHUNKS_W, CHUNK), jnp.int32),
          pltpu.VMEM((CHUNK, LANES), jnp.float32),
          pltpu.VMEM((CHUNK, LANES), jnp.float32),
          pltpu.SemaphoreType.DMA,
          pltpu.SemaphoreType.DMA,
          pltpu.SemaphoreType.DMA,
          pltpu.SemaphoreType.DMA,
          pltpu.VMEM_SHARED((ACC_R, LANES), jnp.float32),
      ],
  )
  return kern(xs16, src_w, dst_w)


def _sc_rows(table, src_r, dst_r):
  kern = pl.kernel(
      _sc_rows_body,
      out_type=jax.ShapeDtypeStruct((NC, ACC_R, HALF), jnp.float32),
      mesh=_mesh(),
      compiler_params=_SC_PARAMS,
      scratch_types=[
          pltpu.VMEM((CHUNKS2, CH2), jnp.int32),
          pltpu.VMEM((CHUNKS2, CH2), jnp.int32),
          pltpu.VMEM((CH2, HALF), jnp.bfloat16),
          pltpu.VMEM((CH2, HALF), jnp.bfloat16),
          pltpu.VMEM((CH2, HALF), jnp.float32),
          pltpu.VMEM((CH2, HALF), jnp.float32),
          pltpu.SemaphoreType.DMA,
          pltpu.SemaphoreType.DMA,
          pltpu.SemaphoreType.DMA,
          pltpu.SemaphoreType.DMA,
          pltpu.VMEM_SHARED((ACC_R, HALF), jnp.float32),
      ],
  )
  return kern(table, src_r, dst_r)


def _dot3(a, b):
  f = jnp.float32
  ahi = a.astype(jnp.bfloat16)
  alo = (a - ahi.astype(f)).astype(jnp.bfloat16)
  bhi = b.astype(jnp.bfloat16)
  blo = (b - bhi.astype(f)).astype(jnp.bfloat16)
  return (jnp.dot(ahi, bhi, preferred_element_type=f)
          + jnp.dot(ahi, blo, preferred_element_type=f)
          + jnp.dot(alo, bhi, preferred_element_type=f))


def _bn_relu(pre, gamma, beta):
  mean = jnp.mean(pre, axis=0, keepdims=True)
  var = jnp.mean((pre - mean) ** 2, axis=0, keepdims=True)
  return jnp.maximum(gamma * (pre - mean) * lax.rsqrt(var + EPS) + beta, 0.0)


def _tc_prep_body(deg2_ref, x_ref, dinv_ref, xs16_ref):
  deg = deg2_ref[0, :N, 0:1] + deg2_ref[1, :N, 0:1]
  dinv = lax.rsqrt(deg)
  dinv_ref[...] = dinv
  xs16_ref[...] = jnp.broadcast_to(dinv * x_ref[...], (N, LANES))


def _split_out(tbl_ref, hsp):
  tbl = hsp.astype(jnp.bfloat16)
  tbl_ref[0] = tbl[:, :HALF]
  tbl_ref[1] = tbl[:, HALF:]


def _tc_layer0_body(s0_ref, dinv_ref, w0_ref, b0_ref, g0_ref,
                    be0_ref, w1p_ref, tbl_ref):
  dinv = dinv_ref[...]
  t = dinv * (s0_ref[0, :N, 0:1] + s0_ref[1, :N, 0:1])
  pre = t * w0_ref[...] + b0_ref[...]
  h = _bn_relu(pre, g0_ref[...], be0_ref[...])
  hsp = dinv * _dot3(h, w1p_ref[...])
  _split_out(tbl_ref, hsp)


def _tc_mid_body(acc_ref, dinv_ref, b_ref, g_ref, be_ref, wp_ref, tbl_ref):
  dinv = dinv_ref[...]
  h_halves = []
  for c in range(NC):
    pre = dinv * acc_ref[c, :N] + b_ref[:, c * HALF:(c + 1) * HALF]
    h_halves.append(_bn_relu(pre, g_ref[:, c * HALF:(c + 1) * HALF],
                             be_ref[:, c * HALF:(c + 1) * HALF]))
  for oc in range(NC):
    osl = slice(oc * HALF, (oc + 1) * HALF)
    hwp = (_dot3(h_halves[0], wp_ref[:HALF, osl])
           + _dot3(h_halves[1], wp_ref[HALF:, osl]))
    tbl_ref[oc] = (dinv * hwp).astype(jnp.bfloat16)


def _tc_final_body(acc_ref, dinv_ref, b_ref, g_ref, be_ref,
                   l1w_ref, l1b_ref, l2w_ref, l2b_ref, out_ref):
  dinv = dinv_ref[...]
  g_halves = []
  for c in range(NC):
    pre = dinv * acc_ref[c, :N] + b_ref[:, c * HALF:(c + 1) * HALF]
    h = _bn_relu(pre, g_ref[:, c * HALF:(c + 1) * HALF],
                 be_ref[:, c * HALF:(c + 1) * HALF])
    g_halves.append(jnp.mean(h, axis=0, keepdims=True))
  g1 = (_dot3(g_halves[0], l1w_ref[:HALF, :])
        + _dot3(g_halves[1], l1w_ref[HALF:, :]))
  g1 = jnp.maximum(g1 + l1b_ref[...], 0.0)
  out_ref[...] = _dot3(g1, l2w_ref[...]) + l2b_ref[...]


def _tc(body, out_shapes, *args):
  return pl.pallas_call(
      body, out_shape=out_shapes,
      compiler_params=pltpu.CompilerParams(vmem_limit_bytes=64 * 1024 * 1024),
  )(*args)


def kernel(x, edge_index, params):
  src = edge_index[0].astype(jnp.int32)
  dst = edge_index[1].astype(jnp.int32)
  loop = jnp.arange(N, dtype=jnp.int32)
  pad = E_PAD - E_TOT
  src_p = jnp.concatenate([src, loop, jnp.zeros((pad,), jnp.int32)])
  dst_p = jnp.concatenate([dst, loop, jnp.full((pad,), N, jnp.int32)])
  src_w = src_p.reshape(NC, NS, CHUNKS_W, CHUNK)
  dst_w = dst_p.reshape(NC, NS, CHUNKS_W, CHUNK)
  src_r = src_p.reshape(NS, CHUNKS2, CH2)
  dst_r = dst_p.reshape(NS, CHUNKS2, CH2)

  convs = params["convs"]
  bns = params["bns"]
  w0 = convs[0]["W"].reshape(1, H)
  bs = [c["b"].reshape(1, H) for c in convs]
  gs = [bn["gamma"].reshape(1, H) for bn in bns]
  bes = [bn["beta"].reshape(1, H) for bn in bns]
  ws = [convs[l]["W"] for l in range(1, 4)]
  wps = [w[:, _PERM] for w in ws]

  deg2 = _sc_hist16(False, jnp.zeros((N, LANES), jnp.float32), src_w, dst_w)
  dinv, xs16 = _tc(
      _tc_prep_body,
      (jax.ShapeDtypeStruct((N, 1), jnp.float32),
       jax.ShapeDtypeStruct((N, LANES), jnp.float32)),
      deg2, x)

  s0 = _sc_hist16(True, xs16, src_w, dst_w)
  tbl = _tc(
      _tc_layer0_body,
      jax.ShapeDtypeStruct((NC, N, HALF), jnp.bfloat16),
      s0, dinv, w0, bs[0], gs[0], bes[0], wps[0])

  for l in (1, 2):
    acc = _sc_rows(tbl, src_r, dst_r)
    tbl = _tc(
        _tc_mid_body,
        jax.ShapeDtypeStruct((NC, N, HALF), jnp.bfloat16),
        acc, dinv, bs[l], gs[l], bes[l], wps[l])

  acc = _sc_rows(tbl, src_r, dst_r)
  out = _tc(
      _tc_final_body,
      jax.ShapeDtypeStruct((1, 1), jnp.float32),
      acc, dinv, bs[3], gs[3], bes[3],
      params["lin1_w"], params["lin1_b"].reshape(1, H),
      params["lin2_w"], params["lin2_b"].reshape(1, 1))
  return out
